# baseline (speedup 1.0000x reference)
import functools

import jax
import jax.numpy as jnp
from jax import lax
from jax.experimental import pallas as pl
from jax.experimental.pallas import tpu as pltpu
from jax.experimental.pallas import tpu_sc as plsc

M, C, D = 65536, 8, 64
B, S, K = 8, 2048, 4
L = 16
UNROLL = 8

NC, NS = 2, 16
NW = NC * NS
DPW = D // NW

BMX = 8192


def _bf16_round(a):
    b = jax.lax.bitcast_convert_type(a, jnp.uint32)
    return (b + jnp.uint32(0x7FFF) + ((b >> 16) & jnp.uint32(1))) >> 16


def _mean_pack_body(st_ref, out_ref):
    acc = st_ref[0]
    for c in range(1, C):
        acc = acc + st_ref[c]
    r = _bf16_round(acc * (1.0 / C))
    w = r[0:D // 2] | (r[D // 2:D] << 16)
    out_ref[...] = jax.lax.bitcast_convert_type(w, jnp.int32)


def _sc_read(idx_hbm, q_hbm, mn_hbm, out_hbm,
             mrow_v, idx0, idx1, q0, q1, o0, o1,
             msem, sem0, sem1, osem0, osem1):
    wid = lax.axis_index("s") * NC + lax.axis_index("c")
    bufs = ((idx0, q0, o0, sem0, osem0), (idx1, q1, o1, sem1, osem1))

    pltpu.async_copy(mn_hbm.at[wid], mrow_v, msem)
    pltpu.make_async_copy(mn_hbm.at[wid], mrow_v, msem).wait()
    himask = jnp.int32(-65536)

    for j in range(DPW):
        d = wid + j * (D // 2)

        def issue(b, d=d):
            idx_v, q_v, _, sem, _ = bufs[b % 2]
            pltpu.async_copy(idx_hbm.at[pl.ds(b * K * S, K * S)], idx_v, sem)
            pltpu.async_copy(q_hbm.at[b, d], q_v, sem)

        issue(0)
        for b in range(B):
            idx_v, q_v, out_v, sem, osem = bufs[b % 2]
            if b + 1 < B:
                issue(b + 1)
            if b >= 2:
                pltpu.make_async_copy(out_v, out_hbm.at[b - 2, d], osem).wait()
            pltpu.make_async_copy(idx_hbm.at[pl.ds(b * K * S, K * S)], idx_v,
                                  sem).wait()
            pltpu.make_async_copy(q_hbm.at[b, d], q_v, sem).wait()

            @plsc.parallel_loop(0, S, step=L, unroll=UNROLL)
            def _svec(s0, idx_v=idx_v, q_v=q_v, out_v=out_v, j=j):
                acc = None
                for k in range(K):
                    iv = idx_v[pl.ds(k * S + s0, L)]
                    g = plsc.load_gather(mrow_v, [iv])
                    if j == 0:
                        bits = g << 16
                    else:
                        bits = g & himask
                    f = plsc.bitcast(bits, jnp.float32)
                    acc = f if acc is None else acc + f
                out_v[pl.ds(s0, L)] = acc * q_v[pl.ds(s0, L)]

            pltpu.async_copy(out_v, out_hbm.at[b, d], osem)

        for b in (B - 2, B - 1):
            _, _, out_v, _, osem = bufs[b % 2]
            pltpu.make_async_copy(out_v, out_hbm.at[b, d], osem).wait()


@jax.jit
def _run(idx1, q_t, states_t):
    packed = pl.pallas_call(
        _mean_pack_body,
        grid=(M // BMX,),
        in_specs=[pl.BlockSpec((C, D, BMX), lambda i: (0, 0, i))],
        out_specs=pl.BlockSpec((D // 2, BMX), lambda i: (0, i)),
        out_shape=jax.ShapeDtypeStruct((D // 2, M), jnp.int32),
    )(states_t)

    f = functools.partial(
        pl.kernel,
        mesh=plsc.VectorSubcoreMesh(core_axis_name="c", subcore_axis_name="s"),
        out_type=jax.ShapeDtypeStruct((B, D, S), jnp.float32),
        scratch_types=[
            pltpu.VMEM((M,), jnp.int32),
            pltpu.VMEM((K * S,), jnp.int32),
            pltpu.VMEM((K * S,), jnp.int32),
            pltpu.VMEM((S,), jnp.float32),
            pltpu.VMEM((S,), jnp.float32),
            pltpu.VMEM((S,), jnp.float32),
            pltpu.VMEM((S,), jnp.float32),
            pltpu.SemaphoreType.DMA,
            pltpu.SemaphoreType.DMA,
            pltpu.SemaphoreType.DMA,
            pltpu.SemaphoreType.DMA,
            pltpu.SemaphoreType.DMA,
        ],
        compiler_params=pltpu.CompilerParams(needs_layout_passes=False),
    )(_sc_read)
    return f(idx1, q_t, packed)


def kernel(partition_indices, queries, states):
    states_t = jnp.transpose(states, (1, 2, 0))
    q_t = jnp.transpose(queries, (0, 2, 1))
    idx1 = jnp.transpose(partition_indices, (0, 2, 1)).reshape(B * K * S)
    idx1 = idx1.astype(jnp.int32)
    out_t = _run(idx1, q_t, states_t)
    return jnp.transpose(out_t, (0, 2, 1))


# speedup vs baseline: 1.0133x; 1.0133x over previous
import functools

import jax
import jax.numpy as jnp
from jax import lax
from jax.experimental import pallas as pl
from jax.experimental.pallas import tpu as pltpu
from jax.experimental.pallas import tpu_sc as plsc

M, C, D = 65536, 8, 64
B, S, K = 8, 2048, 4
L = 16
UNROLL = 8

NC, NS = 2, 16
NW = NC * NS
DPW = D // NW

BMX = 4096


def _bf16_round(a):
    b = jax.lax.bitcast_convert_type(a, jnp.uint32)
    return (b + jnp.uint32(0x7FFF) + ((b >> 16) & jnp.uint32(1))) >> 16


def _mean_pack_body(st_ref, out_ref):
    acc = st_ref[0]
    for c in range(1, C):
        acc = acc + st_ref[c]
    r = _bf16_round(acc * (1.0 / C))
    w = r[0:D // 2] | (r[D // 2:D] << 16)
    out_ref[...] = jax.lax.bitcast_convert_type(w, jnp.int32)


def _sc_read(idx_hbm, q_hbm, mn_hbm, out_hbm,
             mrow_v, idx0, idx1, q0, q1, o0, o1,
             msem, sem0, sem1, osem0, osem1):
    wid = lax.axis_index("s") * NC + lax.axis_index("c")
    bufs = ((idx0, q0, o0, sem0, osem0), (idx1, q1, o1, sem1, osem1))

    pltpu.async_copy(mn_hbm.at[wid], mrow_v, msem)
    pltpu.make_async_copy(mn_hbm.at[wid], mrow_v, msem).wait()
    himask = jnp.int32(-65536)

    for j in range(DPW):
        d = wid + j * (D // 2)

        def issue(b, d=d):
            idx_v, q_v, _, sem, _ = bufs[b % 2]
            pltpu.async_copy(idx_hbm.at[pl.ds(b * K * S, K * S)], idx_v, sem)
            pltpu.async_copy(q_hbm.at[b, d], q_v, sem)

        issue(0)
        for b in range(B):
            idx_v, q_v, out_v, sem, osem = bufs[b % 2]
            if b + 1 < B:
                issue(b + 1)
            if b >= 2:
                pltpu.make_async_copy(out_v, out_hbm.at[b - 2, d], osem).wait()
            pltpu.make_async_copy(idx_hbm.at[pl.ds(b * K * S, K * S)], idx_v,
                                  sem).wait()
            pltpu.make_async_copy(q_hbm.at[b, d], q_v, sem).wait()

            @plsc.parallel_loop(0, S, step=L, unroll=UNROLL)
            def _svec(s0, idx_v=idx_v, q_v=q_v, out_v=out_v, j=j):
                acc = None
                for k in range(K):
                    iv = idx_v[pl.ds(k * S + s0, L)]
                    g = plsc.load_gather(mrow_v, [iv])
                    if j == 0:
                        bits = g << 16
                    else:
                        bits = g & himask
                    f = plsc.bitcast(bits, jnp.float32)
                    acc = f if acc is None else acc + f
                out_v[pl.ds(s0, L)] = acc * q_v[pl.ds(s0, L)]

            pltpu.async_copy(out_v, out_hbm.at[b, d], osem)

        for b in (B - 2, B - 1):
            _, _, out_v, _, osem = bufs[b % 2]
            pltpu.make_async_copy(out_v, out_hbm.at[b, d], osem).wait()


@jax.jit
def _run(idx1, q_t, states_t):
    packed = pl.pallas_call(
        _mean_pack_body,
        grid=(M // BMX,),
        in_specs=[pl.BlockSpec((C, D, BMX), lambda i: (0, 0, i))],
        out_specs=pl.BlockSpec((D // 2, BMX), lambda i: (0, i)),
        out_shape=jax.ShapeDtypeStruct((D // 2, M), jnp.int32),
    )(states_t)

    f = functools.partial(
        pl.kernel,
        mesh=plsc.VectorSubcoreMesh(core_axis_name="c", subcore_axis_name="s"),
        out_type=jax.ShapeDtypeStruct((B, D, S), jnp.float32),
        scratch_types=[
            pltpu.VMEM((M,), jnp.int32),
            pltpu.VMEM((K * S,), jnp.int32),
            pltpu.VMEM((K * S,), jnp.int32),
            pltpu.VMEM((S,), jnp.float32),
            pltpu.VMEM((S,), jnp.float32),
            pltpu.VMEM((S,), jnp.float32),
            pltpu.VMEM((S,), jnp.float32),
            pltpu.SemaphoreType.DMA,
            pltpu.SemaphoreType.DMA,
            pltpu.SemaphoreType.DMA,
            pltpu.SemaphoreType.DMA,
            pltpu.SemaphoreType.DMA,
        ],
        compiler_params=pltpu.CompilerParams(needs_layout_passes=False),
    )(_sc_read)
    return f(idx1, q_t, packed)


def kernel(partition_indices, queries, states):
    states_t = jnp.transpose(states, (1, 2, 0))
    q_t = jnp.transpose(queries, (0, 2, 1))
    idx1 = jnp.transpose(partition_indices, (0, 2, 1)).reshape(B * K * S)
    idx1 = idx1.astype(jnp.int32)
    out_t = _run(idx1, q_t, states_t)
    return jnp.transpose(out_t, (0, 2, 1))
